# baseline (speedup 1.0000x reference)
import functools

import jax
import jax.numpy as jnp
from jax import lax
from jax.experimental import pallas as pl
from jax.experimental.pallas import tpu as pltpu
from jax.experimental.pallas import tpu_sc as plsc

N_NODES = 10000
N_EDGES = 320000
D_FEAT = 128

NUM_CORES = 2
NUM_SUBCORES = 16
NUM_WORKERS = NUM_CORES * NUM_SUBCORES
EDGES_PER_WORKER = N_EDGES // NUM_WORKERS
LANES = 16
UNROLL = 5
WINDOW = EDGES_PER_WORKER + 112


def _tc_table_body(x_ref, w_ref, b_ref, t1_ref, t2_ref):
    nt = (((1,), (1,)), ((), ()))
    t1_ref[...] = lax.dot_general(w_ref[:, :D_FEAT], x_ref[...], nt,
                                  preferred_element_type=jnp.float32) + b_ref[0]
    t2_ref[...] = lax.dot_general(w_ref[:, D_FEAT:], x_ref[...], nt,
                                  preferred_element_type=jnp.float32)


def _make_tables(x, w, b):
    return pl.pallas_call(
        _tc_table_body,
        out_shape=[
            jax.ShapeDtypeStruct((1, N_NODES), jnp.float32),
            jax.ShapeDtypeStruct((1, N_NODES), jnp.float32),
        ],
        in_specs=[
            pl.BlockSpec(memory_space=pltpu.VMEM),
            pl.BlockSpec(memory_space=pltpu.VMEM),
            pl.BlockSpec(memory_space=pltpu.SMEM),
        ],
    )(x, w, b)


@functools.partial(
    pl.kernel,
    mesh=plsc.VectorSubcoreMesh(core_axis_name="c", subcore_axis_name="s"),
    out_type=jax.ShapeDtypeStruct((N_EDGES, 1), jnp.float32),
    compiler_params=pltpu.CompilerParams(
        needs_layout_passes=False, use_tc_tiling_on_sc=False
    ),
    scratch_types=[
        pltpu.VMEM((N_NODES,), jnp.float32),
        pltpu.VMEM((N_NODES,), jnp.float32),
        pltpu.VMEM((2, WINDOW), jnp.int32),
        pltpu.VMEM((EDGES_PER_WORKER, 1), jnp.float32),
        pltpu.SemaphoreType.DMA,
        pltpu.SemaphoreType.DMA,
        pltpu.SemaphoreType.DMA,
    ],
)
def _sc_edge_scores(t1_hbm, t2_hbm, ei_hbm, out_hbm,
                    t1_v, t2_v, sd_v, o_v, sm1, sm2, sm3):
    wid = lax.axis_index("s") * NUM_CORES + lax.axis_index("c")
    base = wid * EDGES_PER_WORKER
    base_al = (base // 128) * 128
    delta = base - base_al

    c1 = pltpu.async_copy(t1_hbm.at[0], t1_v, sm1)
    c2 = pltpu.async_copy(t2_hbm.at[0], t2_v, sm2)
    c3 = pltpu.async_copy(ei_hbm.at[:, pl.ds(base_al, WINDOW)], sd_v, sm3)
    c1.wait()
    c2.wait()
    c3.wait()

    lane = lax.iota(jnp.int32, LANES)
    zero = jnp.zeros((LANES,), jnp.int32)

    @plsc.parallel_loop(0, EDGES_PER_WORKER, LANES, unroll=UNROLL)
    def _(off):
        si = sd_v[0, pl.ds(delta + off, LANES)]
        di = sd_v[1, pl.ds(delta + off, LANES)]
        g1 = plsc.load_gather(t1_v, [si])
        g2 = plsc.load_gather(t2_v, [di])
        plsc.store_scatter(o_v, [off + lane, zero], g1 + g2)

    pltpu.sync_copy(o_v, out_hbm.at[pl.ds(base, EDGES_PER_WORKER)])


def kernel(x, edge_index, W_weight, W_bias):
    t1, t2 = _make_tables(x, W_weight, W_bias)
    return _sc_edge_scores(t1, t2, edge_index.astype(jnp.int32))


# speedup vs baseline: 6.1601x; 6.1601x over previous
import functools

import jax
import jax.numpy as jnp
from jax import lax
from jax.experimental import pallas as pl
from jax.experimental.pallas import tpu as pltpu
from jax.experimental.pallas import tpu_sc as plsc

N_NODES = 10000
N_EDGES = 320000
D_FEAT = 128

NUM_CORES = 2
NUM_SUBCORES = 16
NUM_WORKERS = NUM_CORES * NUM_SUBCORES
EDGES_PER_WORKER = N_EDGES // NUM_WORKERS
LANES = 16
UNROLL = 10
WINDOW = EDGES_PER_WORKER + 112


def _tc_table_body(x_ref, w_ref, b_ref, t1_ref, t2_ref):
    nt = (((1,), (1,)), ((), ()))
    t1_ref[...] = lax.dot_general(w_ref[:, :D_FEAT], x_ref[...], nt,
                                  preferred_element_type=jnp.float32) + b_ref[0]
    t2_ref[...] = lax.dot_general(w_ref[:, D_FEAT:], x_ref[...], nt,
                                  preferred_element_type=jnp.float32)


def _make_tables(x, w, b):
    return pl.pallas_call(
        _tc_table_body,
        out_shape=[
            jax.ShapeDtypeStruct((1, N_NODES), jnp.float32),
            jax.ShapeDtypeStruct((1, N_NODES), jnp.float32),
        ],
        in_specs=[
            pl.BlockSpec(memory_space=pltpu.VMEM),
            pl.BlockSpec(memory_space=pltpu.VMEM),
            pl.BlockSpec(memory_space=pltpu.SMEM),
        ],
    )(x, w, b)


@functools.partial(
    pl.kernel,
    mesh=plsc.VectorSubcoreMesh(core_axis_name="c", subcore_axis_name="s"),
    out_type=jax.ShapeDtypeStruct((N_EDGES,), jnp.float32),
    compiler_params=pltpu.CompilerParams(needs_layout_passes=False),
    scratch_types=[
        pltpu.VMEM((N_NODES,), jnp.float32),
        pltpu.VMEM((N_NODES,), jnp.float32),
        pltpu.VMEM((2, WINDOW), jnp.int32),
        pltpu.VMEM((EDGES_PER_WORKER,), jnp.float32),
        pltpu.SemaphoreType.DMA,
        pltpu.SemaphoreType.DMA,
        pltpu.SemaphoreType.DMA,
    ],
)
def _sc_edge_scores(t1_hbm, t2_hbm, ei_hbm, out_hbm,
                    t1_v, t2_v, sd_v, o_v, sm1, sm2, sm3):
    wid = lax.axis_index("s") * NUM_CORES + lax.axis_index("c")
    base = wid * EDGES_PER_WORKER
    base_al = (base // 128) * 128
    delta = base - base_al

    c1 = pltpu.async_copy(t1_hbm.at[0], t1_v, sm1)
    c2 = pltpu.async_copy(t2_hbm.at[0], t2_v, sm2)
    c3 = pltpu.async_copy(ei_hbm.at[:, pl.ds(base_al, WINDOW)], sd_v, sm3)
    c1.wait()
    c2.wait()
    c3.wait()

    @plsc.parallel_loop(0, EDGES_PER_WORKER, LANES, unroll=UNROLL)
    def _(off):
        si = sd_v[0, pl.ds(delta + off, LANES)]
        di = sd_v[1, pl.ds(delta + off, LANES)]
        g1 = plsc.load_gather(t1_v, [si])
        g2 = plsc.load_gather(t2_v, [di])
        o_v[pl.ds(off, LANES)] = g1 + g2

    pltpu.sync_copy(o_v, out_hbm.at[pl.ds(base, EDGES_PER_WORKER)])


def kernel(x, edge_index, W_weight, W_bias):
    t1, t2 = _make_tables(x, W_weight, W_bias)
    scores = _sc_edge_scores(t1, t2, edge_index.astype(jnp.int32))
    return scores.reshape(N_EDGES, 1)


# speedup vs baseline: 6.1645x; 1.0007x over previous
import functools

import jax
import jax.numpy as jnp
from jax import lax
from jax.experimental import pallas as pl
from jax.experimental.pallas import tpu as pltpu
from jax.experimental.pallas import tpu_sc as plsc

N_NODES = 10000
N_EDGES = 320000
D_FEAT = 128

NUM_CORES = 2
NUM_SUBCORES = 16
NUM_WORKERS = NUM_CORES * NUM_SUBCORES
EDGES_PER_WORKER = N_EDGES // NUM_WORKERS
LANES = 16
UNROLL = 10
WINDOW = EDGES_PER_WORKER + 112


def _tc_table_body(x_ref, w_ref, b_ref, t1_ref, t2_ref):
    nt = (((1,), (1,)), ((), ()))
    t1_ref[...] = lax.dot_general(w_ref[:, :D_FEAT], x_ref[...], nt,
                                  preferred_element_type=jnp.float32) + b_ref[0]
    t2_ref[...] = lax.dot_general(w_ref[:, D_FEAT:], x_ref[...], nt,
                                  preferred_element_type=jnp.float32)


def _make_tables(x, w, b):
    return pl.pallas_call(
        _tc_table_body,
        out_shape=[
            jax.ShapeDtypeStruct((1, N_NODES), jnp.float32),
            jax.ShapeDtypeStruct((1, N_NODES), jnp.float32),
        ],
        in_specs=[
            pl.BlockSpec(memory_space=pltpu.VMEM),
            pl.BlockSpec(memory_space=pltpu.VMEM),
            pl.BlockSpec(memory_space=pltpu.SMEM),
        ],
    )(x, w, b)


@functools.partial(
    pl.kernel,
    mesh=plsc.VectorSubcoreMesh(core_axis_name="c", subcore_axis_name="s"),
    out_type=jax.ShapeDtypeStruct((N_EDGES,), jnp.float32),
    compiler_params=pltpu.CompilerParams(
        needs_layout_passes=False,
        skip_device_barrier=True,
        disable_bounds_checks=True,
        disable_semaphore_checks=True,
    ),
    scratch_types=[
        pltpu.VMEM((N_NODES,), jnp.float32),
        pltpu.VMEM((N_NODES,), jnp.float32),
        pltpu.VMEM((2, WINDOW), jnp.int32),
        pltpu.VMEM((EDGES_PER_WORKER,), jnp.float32),
        pltpu.SemaphoreType.DMA,
        pltpu.SemaphoreType.DMA,
        pltpu.SemaphoreType.DMA,
    ],
)
def _sc_edge_scores(t1_hbm, t2_hbm, ei_hbm, out_hbm,
                    t1_v, t2_v, sd_v, o_v, sm1, sm2, sm3):
    wid = lax.axis_index("s") * NUM_CORES + lax.axis_index("c")
    base = wid * EDGES_PER_WORKER
    base_al = (base // 128) * 128
    delta = base - base_al

    c1 = pltpu.async_copy(t1_hbm.at[0], t1_v, sm1)
    c2 = pltpu.async_copy(t2_hbm.at[0], t2_v, sm2)
    c3 = pltpu.async_copy(ei_hbm.at[:, pl.ds(base_al, WINDOW)], sd_v, sm3)
    c1.wait()
    c2.wait()
    c3.wait()

    @plsc.parallel_loop(0, EDGES_PER_WORKER, LANES, unroll=UNROLL)
    def _(off):
        si = sd_v[0, pl.ds(delta + off, LANES)]
        di = sd_v[1, pl.ds(delta + off, LANES)]
        g1 = plsc.load_gather(t1_v, [si])
        g2 = plsc.load_gather(t2_v, [di])
        o_v[pl.ds(off, LANES)] = g1 + g2

    pltpu.sync_copy(o_v, out_hbm.at[pl.ds(base, EDGES_PER_WORKER)])


def kernel(x, edge_index, W_weight, W_bias):
    t1, t2 = _make_tables(x, W_weight, W_bias)
    scores = _sc_edge_scores(t1, t2, edge_index.astype(jnp.int32))
    return scores.reshape(N_EDGES, 1)
